# baseline (speedup 1.0000x reference)
import functools

import jax
import jax.numpy as jnp
from jax import lax
from jax.experimental import pallas as pl
from jax.experimental.pallas import tpu as pltpu
from jax.experimental.pallas import tpu_sc as plsc

B = 4096
L = 200
D = 64
NC = 2
NS = 16
NW = NC * NS
BPW = B // NW
LANES = 16
NCH = D // LANES


VB = 128
NBLK = VOCAB_MAIN = 999936 // VB
BLK_PER_W = 245


def _format_sc(embT, tail_lin):
    V = 1000000
    mesh = plsc.VectorSubcoreMesh(core_axis_name="c", subcore_axis_name="s")

    @functools.partial(
        pl.kernel,
        mesh=mesh,
        out_type=jax.ShapeDtypeStruct((V * D,), jnp.float32),
    scratch_types=[
            pltpu.VMEM((D, VB), jnp.float32),
            pltpu.VMEM((D, VB), jnp.float32),
            pltpu.VMEM((D * VB,), jnp.float32),
            pltpu.VMEM((D * VB,), jnp.float32),
            pltpu.SemaphoreType.DMA,
            pltpu.SemaphoreType.DMA,
            pltpu.SemaphoreType.DMA,
            pltpu.SemaphoreType.DMA,
        ],
        compiler_params=pltpu.CompilerParams(use_tc_tiling_on_sc=True,
                                             needs_layout_passes=False),
    )
    def k(embT_hbm, tail_hbm, out_hbm, in_v0, in_v1, out_v0, out_v1,
          si0, si1, so0, so1):
        wid = lax.axis_index("s") * NC + lax.axis_index("c")
        base_blk = wid * BLK_PER_W
        ins = (in_v0, in_v1)
        outs = (out_v0, out_v1)
        sin = (si0, si1)
        sout = (so0, so1)
        lane64 = jnp.arange(16, dtype=jnp.int32) * D

        def start_in(t, buf):
            blk = base_blk + t

            @pl.when((blk < NBLK) & (t < BLK_PER_W))
            def _():
                pltpu.make_async_copy(
                    embT_hbm.at[:, pl.ds(blk * VB, VB)], ins[buf],
                    sin[buf]).start()

        def wait_in(buf):
            pltpu.make_async_copy(
                embT_hbm.at[:, pl.ds(0, VB)], ins[buf], sin[buf]).wait()

        def transpose_blk(buf):
            def d_body(d, carry):
                for vc in range(VB // 16):
                    vec = ins[buf][d, pl.ds(vc * 16, 16)]
                    idx = lane64 + (d + vc * 16 * D)
                    plsc.store_scatter(outs[buf], [idx], vec)
                return carry
            lax.fori_loop(0, D, d_body, 0, unroll=2)

        def start_out(t, buf):
            blk = base_blk + t
            pltpu.make_async_copy(
                outs[buf], out_hbm.at[pl.ds(blk * VB * D, VB * D)],
                sout[buf]).start()

        def wait_out(buf):
            pltpu.make_async_copy(
                outs[buf], out_hbm.at[pl.ds(0, VB * D)], sout[buf]).wait()

        start_in(0, 0)
        start_in(1, 1)

        def pair_body(i, carry):
            t = 2 * i
            for buf in (0, 1):
                tt = t + buf

                @pl.when((base_blk + tt < NBLK) & (tt < BLK_PER_W))
                def _():
                    @pl.when(tt >= 2)
                    def _():
                        wait_out(buf)
                    wait_in(buf)
                    transpose_blk(buf)
                    start_out(tt, buf)
                    start_in(tt + 2, buf)
            return carry

        lax.fori_loop(0, (BLK_PER_W + 1) // 2, pair_body, 0)

        @pl.when(base_blk < NBLK)
        def _():
            wait_out(0)

        @pl.when(base_blk + 1 < NBLK)
        def _():
            wait_out(1)

        @pl.when(wid == 0)
        def _():
            pltpu.sync_copy(tail_hbm, out_v0.at[pl.ds(0, 64 * D)])
            pltpu.sync_copy(out_v0.at[pl.ds(0, 64 * D)],
                            out_hbm.at[pl.ds(NBLK * VB * D, 64 * D)])

    return k(embT, tail_lin)


def _pool_sc(x, emb):
    mesh = plsc.VectorSubcoreMesh(core_axis_name="c", subcore_axis_name="s")

    @functools.partial(
        pl.kernel,
        mesh=mesh,
        out_type=jax.ShapeDtypeStruct((B, 2 * D), jnp.float32),
        scratch_types=[
            pltpu.VMEM((BPW, L), jnp.int32),
            pltpu.VMEM((2, L, D), jnp.float32),
            pltpu.VMEM((BPW, 2 * D), jnp.float32),
            pltpu.SemaphoreType.DMA,
            pltpu.SemaphoreType.DMA,
        ],
        compiler_params=pltpu.CompilerParams(use_tc_tiling_on_sc=False),
    )
    def k(x_hbm, emb_hbm, out_hbm, idx_v, rows_v, out_v, sem0, sem1):
        wid = lax.axis_index("s") * NC + lax.axis_index("c")
        base = wid * BPW
        pltpu.sync_copy(x_hbm.at[pl.ds(base, BPW)], idx_v)
        sems = (sem0, sem1)

        def start_gather(row, buf):
            sem = sems[buf]
            pltpu.make_async_copy(
                emb_hbm.at[idx_v.at[row, pl.ds(0, 128)]],
                rows_v.at[buf, pl.ds(0, 128)], sem).start()
            pltpu.make_async_copy(
                emb_hbm.at[idx_v.at[row, pl.ds(128, L - 128)]],
                rows_v.at[buf, pl.ds(128, L - 128)], sem).start()

        def wait_gather(buf):
            sem = sems[buf]
            pltpu.make_async_copy(
                emb_hbm.at[pl.ds(0, 128)], rows_v.at[buf, pl.ds(0, 128)],
                sem).wait()
            pltpu.make_async_copy(
                emb_hbm.at[pl.ds(0, L - 128)],
                rows_v.at[buf, pl.ds(128, L - 128)], sem).wait()

        def reduce_row(row, buf):
            neg = jnp.full((LANES,), -3.4e38, dtype=jnp.float32)
            zero = jnp.zeros((LANES,), dtype=jnp.float32)
            init = (zero, zero, zero, zero, neg, neg, neg, neg)

            def red_body(i, acc):
                s0, s1, s2, s3, m0, m1, m2, m3 = acc
                v0 = rows_v[buf, i, pl.ds(0, LANES)]
                v1 = rows_v[buf, i, pl.ds(LANES, LANES)]
                v2 = rows_v[buf, i, pl.ds(2 * LANES, LANES)]
                v3 = rows_v[buf, i, pl.ds(3 * LANES, LANES)]
                return (s0 + v0, s1 + v1, s2 + v2, s3 + v3,
                        jnp.maximum(m0, v0), jnp.maximum(m1, v1),
                        jnp.maximum(m2, v2), jnp.maximum(m3, v3))

            s0, s1, s2, s3, m0, m1, m2, m3 = lax.fori_loop(
                0, L, red_body, init, unroll=4)
            inv = jnp.float32(1.0 / L)
            out_v[row, pl.ds(0, LANES)] = s0 * inv
            out_v[row, pl.ds(LANES, LANES)] = s1 * inv
            out_v[row, pl.ds(2 * LANES, LANES)] = s2 * inv
            out_v[row, pl.ds(3 * LANES, LANES)] = s3 * inv
            out_v[row, pl.ds(D, LANES)] = m0
            out_v[row, pl.ds(D + LANES, LANES)] = m1
            out_v[row, pl.ds(D + 2 * LANES, LANES)] = m2
            out_v[row, pl.ds(D + 3 * LANES, LANES)] = m3

        start_gather(0, 0)
        start_gather(1, 1)

        def pair_body(i, carry):
            row = 2 * i
            for buf in (0, 1):
                wait_gather(buf)
                reduce_row(row + buf, buf)

                @pl.when(row + buf + 2 < BPW)
                def _():
                    start_gather(row + buf + 2, buf)
            return carry

        lax.fori_loop(0, BPW // 2, pair_body, 0)
        pltpu.sync_copy(out_v, out_hbm.at[pl.ds(base, BPW)])

    return k(x, emb)


def _mlp_tc_body(p_ref, w1_ref, b1_ref, w2_ref, b2_ref, w3_ref, b3_ref,
                 o_ref):
    p = p_ref[...]
    h = lax.dot_general(p, w1_ref[...], (((1,), (1,)), ((), ())),
                        preferred_element_type=jnp.float32)
    h = jnp.maximum(h + b1_ref[...], 0.0)
    h2 = lax.dot_general(h, w2_ref[...], (((1,), (1,)), ((), ())),
                         preferred_element_type=jnp.float32)
    h2 = jnp.maximum(h2 + b2_ref[...], 0.0)
    o = lax.dot_general(h2, w3_ref[...], (((1,), (1,)), ((), ())),
                        preferred_element_type=jnp.float32)
    o_ref[...] = jax.nn.sigmoid(o + b3_ref[...])


def _mlp_tc(pooled, W1, b1, W2, b2, W3, b3):
    W3p = jnp.zeros((128, 64), jnp.float32).at[0].set(W3[0])
    b3p = jnp.zeros((1, 128), jnp.float32).at[0, 0].set(b3[0])
    out = pl.pallas_call(
        _mlp_tc_body,
        out_shape=jax.ShapeDtypeStruct((B, 128), jnp.float32),
    )(pooled, W1, b1.reshape(1, 128), W2, b2.reshape(1, 64), W3p, b3p)
    return out[:, :1]


def kernel(x, emb, W1, b1, W2, b2, W3, b3):
    tail_lin = emb[NBLK * VB:, :].reshape(64 * D)
    emb_lin = _format_sc(emb.T, tail_lin).reshape(1000000, D)
    pooled = _pool_sc(x.astype(jnp.int32), emb_lin)
    return _mlp_tc(pooled, W1, b1, W2, b2, W3, b3)


# speedup vs baseline: 2.3153x; 2.3153x over previous
import functools

import jax
import jax.numpy as jnp
from jax import lax
from jax.experimental import pallas as pl
from jax.experimental.pallas import tpu as pltpu
from jax.experimental.pallas import tpu_sc as plsc

B = 4096
L = 200
D = 64
NC = 2
NS = 16
NW = NC * NS
BPW = B // NW
LANES = 16
NCH = D // LANES


VB = 128
NBLK = VOCAB_MAIN = 999936 // VB
BLK_PER_W = 245


def _format_sc(embT, tail_lin):
    V = 1000000
    mesh = plsc.VectorSubcoreMesh(core_axis_name="c", subcore_axis_name="s")

    @functools.partial(
        pl.kernel,
        mesh=mesh,
        out_type=jax.ShapeDtypeStruct((V * D,), jnp.float32),
    scratch_types=[
            pltpu.VMEM((D, VB), jnp.float32),
            pltpu.VMEM((D, VB), jnp.float32),
            pltpu.VMEM((D * VB,), jnp.float32),
            pltpu.VMEM((D * VB,), jnp.float32),
            pltpu.SemaphoreType.DMA,
            pltpu.SemaphoreType.DMA,
            pltpu.SemaphoreType.DMA,
            pltpu.SemaphoreType.DMA,
        ],
        compiler_params=pltpu.CompilerParams(use_tc_tiling_on_sc=True,
                                             needs_layout_passes=False),
    )
    def k(embT_hbm, tail_hbm, out_hbm, in_v0, in_v1, out_v0, out_v1,
          si0, si1, so0, so1):
        wid = lax.axis_index("s") * NC + lax.axis_index("c")
        base_blk = wid * BLK_PER_W
        ins = (in_v0, in_v1)
        outs = (out_v0, out_v1)
        sin = (si0, si1)
        sout = (so0, so1)
        lane = jnp.arange(16, dtype=jnp.int32)
        perms = [(lane + k) & 15 for k in range(16)]
        outbase = [p * D + lane for p in perms]

        def start_in(t, buf):
            blk = base_blk + t

            @pl.when((blk < NBLK) & (t < BLK_PER_W))
            def _():
                pltpu.make_async_copy(
                    embT_hbm.at[:, pl.ds(blk * VB, VB)], ins[buf],
                    sin[buf]).start()

        def wait_in(buf):
            pltpu.make_async_copy(
                embT_hbm.at[:, pl.ds(0, VB)], ins[buf], sin[buf]).wait()

        def transpose_blk(buf):
            def dg_body(dg, carry):
                d0 = dg * 16
                rows_in = lane + d0
                for vg in range(VB // 16):
                    for k in range(16):
                        cols_in = perms[k] + (vg * 16)
                        vec = plsc.load_gather(ins[buf], [rows_in, cols_in])
                        oidx = outbase[k] + (vg * 16 * D + d0)
                        plsc.store_scatter(outs[buf], [oidx], vec)
                return carry
            lax.fori_loop(0, D // 16, dg_body, 0)

        def start_out(t, buf):
            blk = base_blk + t
            pltpu.make_async_copy(
                outs[buf], out_hbm.at[pl.ds(blk * VB * D, VB * D)],
                sout[buf]).start()

        def wait_out(buf):
            pltpu.make_async_copy(
                outs[buf], out_hbm.at[pl.ds(0, VB * D)], sout[buf]).wait()

        start_in(0, 0)
        start_in(1, 1)

        def pair_body(i, carry):
            t = 2 * i
            for buf in (0, 1):
                tt = t + buf

                @pl.when((base_blk + tt < NBLK) & (tt < BLK_PER_W))
                def _():
                    @pl.when(tt >= 2)
                    def _():
                        wait_out(buf)
                    wait_in(buf)
                    transpose_blk(buf)
                    start_out(tt, buf)
                    start_in(tt + 2, buf)
            return carry

        lax.fori_loop(0, (BLK_PER_W + 1) // 2, pair_body, 0)

        @pl.when(base_blk < NBLK)
        def _():
            wait_out(0)

        @pl.when(base_blk + 1 < NBLK)
        def _():
            wait_out(1)

        @pl.when(wid == 0)
        def _():
            pltpu.sync_copy(tail_hbm, out_v0.at[pl.ds(0, 64 * D)])
            pltpu.sync_copy(out_v0.at[pl.ds(0, 64 * D)],
                            out_hbm.at[pl.ds(NBLK * VB * D, 64 * D)])

    return k(embT, tail_lin)


def _pool_sc(x, emb):
    mesh = plsc.VectorSubcoreMesh(core_axis_name="c", subcore_axis_name="s")

    @functools.partial(
        pl.kernel,
        mesh=mesh,
        out_type=jax.ShapeDtypeStruct((B, 2 * D), jnp.float32),
        scratch_types=[
            pltpu.VMEM((BPW, L), jnp.int32),
            pltpu.VMEM((2, L, D), jnp.float32),
            pltpu.VMEM((BPW, 2 * D), jnp.float32),
            pltpu.SemaphoreType.DMA,
            pltpu.SemaphoreType.DMA,
        ],
        compiler_params=pltpu.CompilerParams(use_tc_tiling_on_sc=False),
    )
    def k(x_hbm, emb_hbm, out_hbm, idx_v, rows_v, out_v, sem0, sem1):
        wid = lax.axis_index("s") * NC + lax.axis_index("c")
        base = wid * BPW
        pltpu.sync_copy(x_hbm.at[pl.ds(base, BPW)], idx_v)
        sems = (sem0, sem1)

        def start_gather(row, buf):
            sem = sems[buf]
            pltpu.make_async_copy(
                emb_hbm.at[idx_v.at[row, pl.ds(0, 128)]],
                rows_v.at[buf, pl.ds(0, 128)], sem).start()
            pltpu.make_async_copy(
                emb_hbm.at[idx_v.at[row, pl.ds(128, L - 128)]],
                rows_v.at[buf, pl.ds(128, L - 128)], sem).start()

        def wait_gather(buf):
            sem = sems[buf]
            pltpu.make_async_copy(
                emb_hbm.at[pl.ds(0, 128)], rows_v.at[buf, pl.ds(0, 128)],
                sem).wait()
            pltpu.make_async_copy(
                emb_hbm.at[pl.ds(0, L - 128)],
                rows_v.at[buf, pl.ds(128, L - 128)], sem).wait()

        def reduce_row(row, buf):
            neg = jnp.full((LANES,), -3.4e38, dtype=jnp.float32)
            zero = jnp.zeros((LANES,), dtype=jnp.float32)
            init = (zero, zero, zero, zero, neg, neg, neg, neg)

            def red_body(i, acc):
                s0, s1, s2, s3, m0, m1, m2, m3 = acc
                v0 = rows_v[buf, i, pl.ds(0, LANES)]
                v1 = rows_v[buf, i, pl.ds(LANES, LANES)]
                v2 = rows_v[buf, i, pl.ds(2 * LANES, LANES)]
                v3 = rows_v[buf, i, pl.ds(3 * LANES, LANES)]
                return (s0 + v0, s1 + v1, s2 + v2, s3 + v3,
                        jnp.maximum(m0, v0), jnp.maximum(m1, v1),
                        jnp.maximum(m2, v2), jnp.maximum(m3, v3))

            s0, s1, s2, s3, m0, m1, m2, m3 = lax.fori_loop(
                0, L, red_body, init, unroll=4)
            inv = jnp.float32(1.0 / L)
            out_v[row, pl.ds(0, LANES)] = s0 * inv
            out_v[row, pl.ds(LANES, LANES)] = s1 * inv
            out_v[row, pl.ds(2 * LANES, LANES)] = s2 * inv
            out_v[row, pl.ds(3 * LANES, LANES)] = s3 * inv
            out_v[row, pl.ds(D, LANES)] = m0
            out_v[row, pl.ds(D + LANES, LANES)] = m1
            out_v[row, pl.ds(D + 2 * LANES, LANES)] = m2
            out_v[row, pl.ds(D + 3 * LANES, LANES)] = m3

        start_gather(0, 0)
        start_gather(1, 1)

        def pair_body(i, carry):
            row = 2 * i
            for buf in (0, 1):
                wait_gather(buf)
                reduce_row(row + buf, buf)

                @pl.when(row + buf + 2 < BPW)
                def _():
                    start_gather(row + buf + 2, buf)
            return carry

        lax.fori_loop(0, BPW // 2, pair_body, 0)
        pltpu.sync_copy(out_v, out_hbm.at[pl.ds(base, BPW)])

    return k(x, emb)


def _mlp_tc_body(p_ref, w1_ref, b1_ref, w2_ref, b2_ref, w3_ref, b3_ref,
                 o_ref):
    p = p_ref[...]
    h = lax.dot_general(p, w1_ref[...], (((1,), (1,)), ((), ())),
                        preferred_element_type=jnp.float32)
    h = jnp.maximum(h + b1_ref[...], 0.0)
    h2 = lax.dot_general(h, w2_ref[...], (((1,), (1,)), ((), ())),
                         preferred_element_type=jnp.float32)
    h2 = jnp.maximum(h2 + b2_ref[...], 0.0)
    o = lax.dot_general(h2, w3_ref[...], (((1,), (1,)), ((), ())),
                        preferred_element_type=jnp.float32)
    o_ref[...] = jax.nn.sigmoid(o + b3_ref[...])


def _mlp_tc(pooled, W1, b1, W2, b2, W3, b3):
    W3p = jnp.zeros((128, 64), jnp.float32).at[0].set(W3[0])
    b3p = jnp.zeros((1, 128), jnp.float32).at[0, 0].set(b3[0])
    out = pl.pallas_call(
        _mlp_tc_body,
        out_shape=jax.ShapeDtypeStruct((B, 128), jnp.float32),
    )(pooled, W1, b1.reshape(1, 128), W2, b2.reshape(1, 64), W3p, b3p)
    return out[:, :1]


def kernel(x, emb, W1, b1, W2, b2, W3, b3):
    tail_lin = emb[NBLK * VB:, :].reshape(64 * D)
    emb_lin = _format_sc(emb.T, tail_lin).reshape(1000000, D)
    pooled = _pool_sc(x.astype(jnp.int32), emb_lin)
    return _mlp_tc(pooled, W1, b1, W2, b2, W3, b3)


# speedup vs baseline: 3.4216x; 1.4778x over previous
import functools

import jax
import jax.numpy as jnp
from jax import lax
from jax.experimental import pallas as pl
from jax.experimental.pallas import tpu as pltpu
from jax.experimental.pallas import tpu_sc as plsc

B = 4096
L = 200
D = 64
NC = 2
NS = 16
NW = NC * NS
BPW = B // NW
LANES = 16
NCH = D // LANES


VB = 128
NBLK = VOCAB_MAIN = 999936 // VB
BLK_PER_W = 245


def _format_sc(embT, tail_lin):
    V = 1000000
    mesh = plsc.VectorSubcoreMesh(core_axis_name="c", subcore_axis_name="s")

    @functools.partial(
        pl.kernel,
        mesh=mesh,
        out_type=jax.ShapeDtypeStruct((V * D,), jnp.float32),
    scratch_types=[
            pltpu.VMEM((D, VB), jnp.float32),
            pltpu.VMEM((D, VB), jnp.float32),
            pltpu.VMEM((D * VB,), jnp.float32),
            pltpu.VMEM((D * VB,), jnp.float32),
            pltpu.SemaphoreType.DMA,
            pltpu.SemaphoreType.DMA,
            pltpu.SemaphoreType.DMA,
            pltpu.SemaphoreType.DMA,
        ],
        compiler_params=pltpu.CompilerParams(use_tc_tiling_on_sc=True,
                                             needs_layout_passes=False),
    )
    def k(embT_hbm, tail_hbm, out_hbm, in_v0, in_v1, out_v0, out_v1,
          si0, si1, so0, so1):
        wid = lax.axis_index("s") * NC + lax.axis_index("c")
        base_blk = wid * BLK_PER_W
        ins = (in_v0, in_v1)
        outs = (out_v0, out_v1)
        sin = (si0, si1)
        sout = (so0, so1)
        lane = jnp.arange(16, dtype=jnp.int32)
        perms = [(lane + k) & 15 for k in range(16)]
        outbase = [p * D + lane for p in perms]

        def start_in(t, buf):
            blk = base_blk + t

            @pl.when((blk < NBLK) & (t < BLK_PER_W))
            def _():
                pltpu.make_async_copy(
                    embT_hbm.at[:, pl.ds(blk * VB, VB)], ins[buf],
                    sin[buf]).start()

        def wait_in(buf):
            pltpu.make_async_copy(
                embT_hbm.at[:, pl.ds(0, VB)], ins[buf], sin[buf]).wait()

        def transpose_blk(buf):
            def dg_body(dg, carry):
                ln, pr, ob = carry[0], carry[1:17], carry[17:]
                d0 = dg * 16
                rows_in = ln + d0
                for vg in range(VB // 16):
                    vecs = [
                        plsc.load_gather(ins[buf],
                                         [rows_in, pr[k] + (vg * 16)])
                        for k in range(16)
                    ]
                    for k in range(16):
                        plsc.store_scatter(
                            outs[buf], [ob[k] + (vg * 16 * D + d0)], vecs[k])
                return carry
            lax.fori_loop(0, D // 16, dg_body,
                          (lane, *perms, *outbase))

        def start_out(t, buf):
            blk = base_blk + t
            pltpu.make_async_copy(
                outs[buf], out_hbm.at[pl.ds(blk * VB * D, VB * D)],
                sout[buf]).start()

        def wait_out(buf):
            pltpu.make_async_copy(
                outs[buf], out_hbm.at[pl.ds(0, VB * D)], sout[buf]).wait()

        start_in(0, 0)
        start_in(1, 1)

        def pair_body(i, carry):
            t = 2 * i
            for buf in (0, 1):
                tt = t + buf

                @pl.when((base_blk + tt < NBLK) & (tt < BLK_PER_W))
                def _():
                    @pl.when(tt >= 2)
                    def _():
                        wait_out(buf)
                    wait_in(buf)
                    transpose_blk(buf)
                    start_out(tt, buf)
                    start_in(tt + 2, buf)
            return carry

        lax.fori_loop(0, (BLK_PER_W + 1) // 2, pair_body, 0)

        @pl.when(base_blk < NBLK)
        def _():
            wait_out(0)

        @pl.when(base_blk + 1 < NBLK)
        def _():
            wait_out(1)

        @pl.when(wid == 0)
        def _():
            pltpu.sync_copy(tail_hbm, out_v0.at[pl.ds(0, 64 * D)])
            pltpu.sync_copy(out_v0.at[pl.ds(0, 64 * D)],
                            out_hbm.at[pl.ds(NBLK * VB * D, 64 * D)])

    return k(embT, tail_lin)


def _pool_sc(x, emb):
    mesh = plsc.VectorSubcoreMesh(core_axis_name="c", subcore_axis_name="s")

    @functools.partial(
        pl.kernel,
        mesh=mesh,
        out_type=jax.ShapeDtypeStruct((B, 2 * D), jnp.float32),
        scratch_types=[
            pltpu.VMEM((BPW, L), jnp.int32),
            pltpu.VMEM((2, L, D), jnp.float32),
            pltpu.VMEM((BPW, 2 * D), jnp.float32),
            pltpu.SemaphoreType.DMA,
            pltpu.SemaphoreType.DMA,
        ],
        compiler_params=pltpu.CompilerParams(use_tc_tiling_on_sc=False),
    )
    def k(x_hbm, emb_hbm, out_hbm, idx_v, rows_v, out_v, sem0, sem1):
        wid = lax.axis_index("s") * NC + lax.axis_index("c")
        base = wid * BPW
        pltpu.sync_copy(x_hbm.at[pl.ds(base, BPW)], idx_v)
        sems = (sem0, sem1)

        def start_gather(row, buf):
            sem = sems[buf]
            pltpu.make_async_copy(
                emb_hbm.at[idx_v.at[row, pl.ds(0, 128)]],
                rows_v.at[buf, pl.ds(0, 128)], sem).start()
            pltpu.make_async_copy(
                emb_hbm.at[idx_v.at[row, pl.ds(128, L - 128)]],
                rows_v.at[buf, pl.ds(128, L - 128)], sem).start()

        def wait_gather(buf):
            sem = sems[buf]
            pltpu.make_async_copy(
                emb_hbm.at[pl.ds(0, 128)], rows_v.at[buf, pl.ds(0, 128)],
                sem).wait()
            pltpu.make_async_copy(
                emb_hbm.at[pl.ds(0, L - 128)],
                rows_v.at[buf, pl.ds(128, L - 128)], sem).wait()

        def reduce_row(row, buf):
            neg = jnp.full((LANES,), -3.4e38, dtype=jnp.float32)
            zero = jnp.zeros((LANES,), dtype=jnp.float32)
            init = (zero, zero, zero, zero, neg, neg, neg, neg)

            def red_body(i, acc):
                s0, s1, s2, s3, m0, m1, m2, m3 = acc
                v0 = rows_v[buf, i, pl.ds(0, LANES)]
                v1 = rows_v[buf, i, pl.ds(LANES, LANES)]
                v2 = rows_v[buf, i, pl.ds(2 * LANES, LANES)]
                v3 = rows_v[buf, i, pl.ds(3 * LANES, LANES)]
                return (s0 + v0, s1 + v1, s2 + v2, s3 + v3,
                        jnp.maximum(m0, v0), jnp.maximum(m1, v1),
                        jnp.maximum(m2, v2), jnp.maximum(m3, v3))

            s0, s1, s2, s3, m0, m1, m2, m3 = lax.fori_loop(
                0, L, red_body, init, unroll=4)
            inv = jnp.float32(1.0 / L)
            out_v[row, pl.ds(0, LANES)] = s0 * inv
            out_v[row, pl.ds(LANES, LANES)] = s1 * inv
            out_v[row, pl.ds(2 * LANES, LANES)] = s2 * inv
            out_v[row, pl.ds(3 * LANES, LANES)] = s3 * inv
            out_v[row, pl.ds(D, LANES)] = m0
            out_v[row, pl.ds(D + LANES, LANES)] = m1
            out_v[row, pl.ds(D + 2 * LANES, LANES)] = m2
            out_v[row, pl.ds(D + 3 * LANES, LANES)] = m3

        start_gather(0, 0)
        start_gather(1, 1)

        def pair_body(i, carry):
            row = 2 * i
            for buf in (0, 1):
                wait_gather(buf)
                reduce_row(row + buf, buf)

                @pl.when(row + buf + 2 < BPW)
                def _():
                    start_gather(row + buf + 2, buf)
            return carry

        lax.fori_loop(0, BPW // 2, pair_body, 0)
        pltpu.sync_copy(out_v, out_hbm.at[pl.ds(base, BPW)])

    return k(x, emb)


def _mlp_tc_body(p_ref, w1_ref, b1_ref, w2_ref, b2_ref, w3_ref, b3_ref,
                 o_ref):
    p = p_ref[...]
    h = lax.dot_general(p, w1_ref[...], (((1,), (1,)), ((), ())),
                        preferred_element_type=jnp.float32)
    h = jnp.maximum(h + b1_ref[...], 0.0)
    h2 = lax.dot_general(h, w2_ref[...], (((1,), (1,)), ((), ())),
                         preferred_element_type=jnp.float32)
    h2 = jnp.maximum(h2 + b2_ref[...], 0.0)
    o = lax.dot_general(h2, w3_ref[...], (((1,), (1,)), ((), ())),
                        preferred_element_type=jnp.float32)
    o_ref[...] = jax.nn.sigmoid(o + b3_ref[...])


def _mlp_tc(pooled, W1, b1, W2, b2, W3, b3):
    W3p = jnp.zeros((128, 64), jnp.float32).at[0].set(W3[0])
    b3p = jnp.zeros((1, 128), jnp.float32).at[0, 0].set(b3[0])
    out = pl.pallas_call(
        _mlp_tc_body,
        out_shape=jax.ShapeDtypeStruct((B, 128), jnp.float32),
    )(pooled, W1, b1.reshape(1, 128), W2, b2.reshape(1, 64), W3p, b3p)
    return out[:, :1]


def kernel(x, emb, W1, b1, W2, b2, W3, b3):
    tail_lin = emb[NBLK * VB:, :].reshape(64 * D)
    emb_lin = _format_sc(emb.T, tail_lin).reshape(1000000, D)
    pooled = _pool_sc(x.astype(jnp.int32), emb_lin)
    return _mlp_tc(pooled, W1, b1, W2, b2, W3, b3)


# speedup vs baseline: 3.8247x; 1.1178x over previous
import functools

import jax
import jax.numpy as jnp
from jax import lax
from jax.experimental import pallas as pl
from jax.experimental.pallas import tpu as pltpu
from jax.experimental.pallas import tpu_sc as plsc

B = 4096
L = 200
D = 64
NC = 2
NS = 16
NW = NC * NS
BPW = B // NW
LANES = 16
NCH = D // LANES


VB = 128
NBLK = VOCAB_MAIN = 999936 // VB
BLK_PER_W = 245


DW = D // 2


def _format_sc(embT, tail_lin):
    V = 1000000
    mesh = plsc.VectorSubcoreMesh(core_axis_name="c", subcore_axis_name="s")

    @functools.partial(
        pl.kernel,
        mesh=mesh,
        out_type=jax.ShapeDtypeStruct((V * DW,), jnp.int32),
        scratch_types=[
            pltpu.VMEM((D, VB), jnp.float32),
            pltpu.VMEM((D, VB), jnp.float32),
            pltpu.VMEM((DW * VB,), jnp.int32),
            pltpu.VMEM((DW * VB,), jnp.int32),
            pltpu.VMEM((64 * D,), jnp.float32),
            pltpu.SemaphoreType.DMA,
            pltpu.SemaphoreType.DMA,
            pltpu.SemaphoreType.DMA,
            pltpu.SemaphoreType.DMA,
        ],
        compiler_params=pltpu.CompilerParams(use_tc_tiling_on_sc=True,
                                             needs_layout_passes=False),
    )
    def k(embT_hbm, tail_hbm, out_hbm, in_v0, in_v1, out_v0, out_v1,
          tail_v, si0, si1, so0, so1):
        wid = lax.axis_index("s") * NC + lax.axis_index("c")
        base_blk = wid * BLK_PER_W
        ins = (in_v0, in_v1)
        outs = (out_v0, out_v1)
        sin = (si0, si1)
        sout = (so0, so1)
        lane = jnp.arange(16, dtype=jnp.int32)
        perms = [(lane + k) & 15 for k in range(16)]
        perms2 = [p * 2 for p in perms]
        outbase = [p + lane * DW for p in perms]

        def start_in(t, buf):
            blk = base_blk + t

            @pl.when((blk < NBLK) & (t < BLK_PER_W))
            def _():
                pltpu.make_async_copy(
                    embT_hbm.at[:, pl.ds(blk * VB, VB)], ins[buf],
                    sin[buf]).start()

        def wait_in(buf):
            pltpu.make_async_copy(
                embT_hbm.at[:, pl.ds(0, VB)], ins[buf], sin[buf]).wait()

        def transpose_blk(buf):
            def vg_body(vg, carry):
                ln, pr2, ob = carry[0], carry[1:17], carry[17:]
                cols = ln + vg * 16
                for jg in range(2):
                    rows = [pr2[kk] + (32 * jg) for kk in range(16)]
                    va = [plsc.load_gather(ins[buf], [rows[kk], cols])
                          for kk in range(16)]
                    vb = [plsc.load_gather(ins[buf], [rows[kk] + 1, cols])
                          for kk in range(16)]
                    for kk in range(16):
                        w = plsc.bitcast(
                            plsc.pack(va[kk], vb[kk],
                                      format=plsc.PackFormat.INTERLEAVED),
                            jnp.int32)
                        plsc.store_scatter(
                            outs[buf],
                            [ob[kk] + (vg * 16 * DW + 16 * jg)], w)
                return carry
            lax.fori_loop(0, VB // 16, vg_body, (lane, *perms2, *outbase))

        def start_out(t, buf):
            blk = base_blk + t
            pltpu.make_async_copy(
                outs[buf], out_hbm.at[pl.ds(blk * VB * DW, VB * DW)],
                sout[buf]).start()

        def wait_out(buf):
            pltpu.make_async_copy(
                outs[buf], out_hbm.at[pl.ds(0, VB * DW)], sout[buf]).wait()

        start_in(0, 0)
        start_in(1, 1)

        def pair_body(i, carry):
            t = 2 * i
            for buf in (0, 1):
                tt = t + buf

                @pl.when((base_blk + tt < NBLK) & (tt < BLK_PER_W))
                def _():
                    @pl.when(tt >= 2)
                    def _():
                        wait_out(buf)
                    wait_in(buf)
                    transpose_blk(buf)
                    start_out(tt, buf)
                    start_in(tt + 2, buf)
            return carry

        lax.fori_loop(0, (BLK_PER_W + 1) // 2, pair_body, 0)

        @pl.when(base_blk < NBLK)
        def _():
            wait_out(0)

        @pl.when(base_blk + 1 < NBLK)
        def _():
            wait_out(1)

        @pl.when(wid == 0)
        def _():
            pltpu.sync_copy(tail_hbm, tail_v)

            def tvg_body(vg, carry):
                for jg in range(2):
                    for kk in range(16):
                        idx_a = lane * D + (vg * 16 * D + 32 * jg
                                            ) + perms2[kk]
                        va = plsc.load_gather(tail_v, [idx_a])
                        vb = plsc.load_gather(tail_v, [idx_a + 1])
                        w = plsc.bitcast(
                            plsc.pack(va, vb,
                                      format=plsc.PackFormat.INTERLEAVED),
                            jnp.int32)
                        oidx = lane * DW + perms[kk] + (vg * 16 * DW
                                                        + 16 * jg)
                        plsc.store_scatter(out_v0, [oidx], w)
                return carry
            lax.fori_loop(0, 4, tvg_body, 0)
            pltpu.sync_copy(out_v0.at[pl.ds(0, 64 * DW)],
                            out_hbm.at[pl.ds(NBLK * VB * DW, 64 * DW)])

    return k(embT, tail_lin)


def _pool_sc(x, emb):
    mesh = plsc.VectorSubcoreMesh(core_axis_name="c", subcore_axis_name="s")

    @functools.partial(
        pl.kernel,
        mesh=mesh,
        out_type=jax.ShapeDtypeStruct((B, 2 * D), jnp.float32),
        scratch_types=[
            pltpu.VMEM((BPW, L), jnp.int32),
            pltpu.VMEM((2, L, DW), jnp.int32),
            pltpu.VMEM((BPW, 2 * D), jnp.float32),
            pltpu.SemaphoreType.DMA,
            pltpu.SemaphoreType.DMA,
        ],
        compiler_params=pltpu.CompilerParams(use_tc_tiling_on_sc=False,
                                             needs_layout_passes=False),
    )
    def k(x_hbm, emb_hbm, out_hbm, idx_v, rows_v, out_v, sem0, sem1):
        wid = lax.axis_index("s") * NC + lax.axis_index("c")
        base = wid * BPW
        pltpu.sync_copy(x_hbm.at[pl.ds(base, BPW)], idx_v)
        sems = (sem0, sem1)

        def start_gather(row, buf):
            sem = sems[buf]
            pltpu.make_async_copy(
                emb_hbm.at[idx_v.at[row, pl.ds(0, 128)]],
                rows_v.at[buf, pl.ds(0, 128)], sem).start()
            pltpu.make_async_copy(
                emb_hbm.at[idx_v.at[row, pl.ds(128, L - 128)]],
                rows_v.at[buf, pl.ds(128, L - 128)], sem).start()

        def wait_gather(buf):
            sem = sems[buf]
            pltpu.make_async_copy(
                emb_hbm.at[pl.ds(0, 128)], rows_v.at[buf, pl.ds(0, 128)],
                sem).wait()
            pltpu.make_async_copy(
                emb_hbm.at[pl.ds(0, L - 128)],
                rows_v.at[buf, pl.ds(128, L - 128)], sem).wait()

        def reduce_row(row, buf):
            neg = jnp.full((LANES,), -3.4e38, dtype=jnp.float32)
            zero = jnp.zeros((LANES,), dtype=jnp.float32)
            init = (zero, zero, zero, zero, neg, neg, neg, neg)

            def red_body(i, acc):
                s0, s1, s2, s3, m0, m1, m2, m3 = acc
                w0 = rows_v[buf, i, pl.ds(0, LANES)]
                w1 = rows_v[buf, i, pl.ds(LANES, LANES)]
                v0, v1 = plsc.unpack(plsc.bitcast(w0, jnp.bfloat16),
                                     format=plsc.PackFormat.INTERLEAVED,
                                     preferred_element_type=jnp.float32)
                v2, v3 = plsc.unpack(plsc.bitcast(w1, jnp.bfloat16),
                                     format=plsc.PackFormat.INTERLEAVED,
                                     preferred_element_type=jnp.float32)
                return (s0 + v0, s1 + v1, s2 + v2, s3 + v3,
                        jnp.maximum(m0, v0), jnp.maximum(m1, v1),
                        jnp.maximum(m2, v2), jnp.maximum(m3, v3))

            s0, s1, s2, s3, m0, m1, m2, m3 = lax.fori_loop(
                0, L, red_body, init, unroll=4)
            inv = jnp.float32(1.0 / L)
            out_v[row, pl.ds(0, LANES)] = s0 * inv
            out_v[row, pl.ds(LANES, LANES)] = s1 * inv
            out_v[row, pl.ds(2 * LANES, LANES)] = s2 * inv
            out_v[row, pl.ds(3 * LANES, LANES)] = s3 * inv
            out_v[row, pl.ds(D, LANES)] = m0
            out_v[row, pl.ds(D + LANES, LANES)] = m1
            out_v[row, pl.ds(D + 2 * LANES, LANES)] = m2
            out_v[row, pl.ds(D + 3 * LANES, LANES)] = m3

        start_gather(0, 0)
        start_gather(1, 1)

        def pair_body(i, carry):
            row = 2 * i
            for buf in (0, 1):
                wait_gather(buf)
                reduce_row(row + buf, buf)

                @pl.when(row + buf + 2 < BPW)
                def _():
                    start_gather(row + buf + 2, buf)
            return carry

        lax.fori_loop(0, BPW // 2, pair_body, 0)
        pltpu.sync_copy(out_v, out_hbm.at[pl.ds(base, BPW)])

    return k(x, emb)


def _mlp_tc_body(p_ref, w1_ref, b1_ref, w2_ref, b2_ref, w3_ref, b3_ref,
                 o_ref):
    p = p_ref[...]
    h = lax.dot_general(p, w1_ref[...], (((1,), (1,)), ((), ())),
                        preferred_element_type=jnp.float32)
    h = jnp.maximum(h + b1_ref[...], 0.0)
    h2 = lax.dot_general(h, w2_ref[...], (((1,), (1,)), ((), ())),
                         preferred_element_type=jnp.float32)
    h2 = jnp.maximum(h2 + b2_ref[...], 0.0)
    o = lax.dot_general(h2, w3_ref[...], (((1,), (1,)), ((), ())),
                        preferred_element_type=jnp.float32)
    o_ref[...] = jax.nn.sigmoid(o + b3_ref[...])


def _mlp_tc(pooled, W1, b1, W2, b2, W3, b3):
    W3p = jnp.zeros((128, 64), jnp.float32).at[0].set(W3[0])
    b3p = jnp.zeros((1, 128), jnp.float32).at[0, 0].set(b3[0])
    out = pl.pallas_call(
        _mlp_tc_body,
        out_shape=jax.ShapeDtypeStruct((B, 128), jnp.float32),
    )(pooled, W1, b1.reshape(1, 128), W2, b2.reshape(1, 64), W3p, b3p)
    return out[:, :1]


import numpy as _np

_PERM64 = _np.concatenate([_np.arange(0, 32, 2), _np.arange(1, 32, 2),
                           _np.arange(32, 64, 2), _np.arange(33, 64, 2)])
_PERM128 = _np.concatenate([_PERM64, _PERM64 + 64])


def kernel(x, emb, W1, b1, W2, b2, W3, b3):
    tail_lin = emb[NBLK * VB:, :].reshape(64 * D)
    emb_lin = _format_sc(emb.T, tail_lin).reshape(1000000, DW)
    pooled = _pool_sc(x.astype(jnp.int32), emb_lin)
    return _mlp_tc(pooled, W1[:, _PERM128], b1, W2, b2, W3, b3)


# speedup vs baseline: 3.8593x; 1.0091x over previous
import functools

import jax
import jax.numpy as jnp
from jax import lax
from jax.experimental import pallas as pl
from jax.experimental.pallas import tpu as pltpu
from jax.experimental.pallas import tpu_sc as plsc

B = 4096
L = 200
D = 64
NC = 2
NS = 16
NW = NC * NS
BPW = B // NW
LANES = 16
NCH = D // LANES


VB = 128
NBLK = VOCAB_MAIN = 999936 // VB
BLK_PER_W = 245


DW = D // 2


def _format_sc(embT, tail_lin):
    V = 1000000
    mesh = plsc.VectorSubcoreMesh(core_axis_name="c", subcore_axis_name="s")

    @functools.partial(
        pl.kernel,
        mesh=mesh,
        out_type=jax.ShapeDtypeStruct((V * DW,), jnp.int32),
        scratch_types=[
            pltpu.VMEM((D, VB), jnp.float32),
            pltpu.VMEM((D, VB), jnp.float32),
            pltpu.VMEM((DW * VB,), jnp.int32),
            pltpu.VMEM((DW * VB,), jnp.int32),
            pltpu.VMEM((64 * D,), jnp.float32),
            pltpu.SemaphoreType.DMA,
            pltpu.SemaphoreType.DMA,
            pltpu.SemaphoreType.DMA,
            pltpu.SemaphoreType.DMA,
        ],
        compiler_params=pltpu.CompilerParams(use_tc_tiling_on_sc=True,
                                             needs_layout_passes=False),
    )
    def k(embT_hbm, tail_hbm, out_hbm, in_v0, in_v1, out_v0, out_v1,
          tail_v, si0, si1, so0, so1):
        wid = lax.axis_index("s") * NC + lax.axis_index("c")
        base_blk = wid * BLK_PER_W
        ins = (in_v0, in_v1)
        outs = (out_v0, out_v1)
        sin = (si0, si1)
        sout = (so0, so1)
        lane = jnp.arange(16, dtype=jnp.int32)
        perms = [(lane + k) & 15 for k in range(16)]
        perms2 = [p * 2 for p in perms]
        outbase = [p + lane * DW for p in perms]

        def start_in(t, buf):
            blk = base_blk + t

            @pl.when((blk < NBLK) & (t < BLK_PER_W))
            def _():
                pltpu.make_async_copy(
                    embT_hbm.at[:, pl.ds(blk * VB, VB)], ins[buf],
                    sin[buf]).start()

        def wait_in(buf):
            pltpu.make_async_copy(
                embT_hbm.at[:, pl.ds(0, VB)], ins[buf], sin[buf]).wait()

        def transpose_blk(buf):
            def vg_body(vg, carry):
                ln, pr2, ob = carry[0], carry[1:17], carry[17:]
                cols = ln + vg * 16
                for jg in range(2):
                    rows = [pr2[kk] + (32 * jg) for kk in range(16)]
                    va = [plsc.load_gather(ins[buf], [rows[kk], cols])
                          for kk in range(16)]
                    vb = [plsc.load_gather(ins[buf], [rows[kk] + 1, cols])
                          for kk in range(16)]
                    for kk in range(16):
                        w = plsc.bitcast(
                            plsc.pack(va[kk], vb[kk],
                                      format=plsc.PackFormat.INTERLEAVED),
                            jnp.int32)
                        plsc.store_scatter(
                            outs[buf],
                            [ob[kk] + (vg * 16 * DW + 16 * jg)], w)
                return carry
            lax.fori_loop(0, VB // 16, vg_body, (lane, *perms2, *outbase))

        def start_out(t, buf):
            blk = base_blk + t
            pltpu.make_async_copy(
                outs[buf], out_hbm.at[pl.ds(blk * VB * DW, VB * DW)],
                sout[buf]).start()

        def wait_out(buf):
            pltpu.make_async_copy(
                outs[buf], out_hbm.at[pl.ds(0, VB * DW)], sout[buf]).wait()

        start_in(0, 0)
        start_in(1, 1)

        def pair_body(i, carry):
            t = 2 * i
            for buf in (0, 1):
                tt = t + buf

                @pl.when((base_blk + tt < NBLK) & (tt < BLK_PER_W))
                def _():
                    @pl.when(tt >= 2)
                    def _():
                        wait_out(buf)
                    wait_in(buf)
                    transpose_blk(buf)
                    start_out(tt, buf)
                    start_in(tt + 2, buf)
            return carry

        lax.fori_loop(0, (BLK_PER_W + 1) // 2, pair_body, 0)

        @pl.when(base_blk < NBLK)
        def _():
            wait_out(0)

        @pl.when(base_blk + 1 < NBLK)
        def _():
            wait_out(1)

        @pl.when(wid == 0)
        def _():
            pltpu.sync_copy(tail_hbm, tail_v)

            def tvg_body(vg, carry):
                for jg in range(2):
                    for kk in range(16):
                        idx_a = lane * D + (vg * 16 * D + 32 * jg
                                            ) + perms2[kk]
                        va = plsc.load_gather(tail_v, [idx_a])
                        vb = plsc.load_gather(tail_v, [idx_a + 1])
                        w = plsc.bitcast(
                            plsc.pack(va, vb,
                                      format=plsc.PackFormat.INTERLEAVED),
                            jnp.int32)
                        oidx = lane * DW + perms[kk] + (vg * 16 * DW
                                                        + 16 * jg)
                        plsc.store_scatter(out_v0, [oidx], w)
                return carry
            lax.fori_loop(0, 4, tvg_body, 0)
            pltpu.sync_copy(out_v0.at[pl.ds(0, 64 * DW)],
                            out_hbm.at[pl.ds(NBLK * VB * DW, 64 * DW)])

    return k(embT, tail_lin)


def _pool_sc(x, emb):
    mesh = plsc.VectorSubcoreMesh(core_axis_name="c", subcore_axis_name="s")

    @functools.partial(
        pl.kernel,
        mesh=mesh,
        out_type=jax.ShapeDtypeStruct((B, 2 * D), jnp.float32),
        scratch_types=[
            pltpu.VMEM((BPW, L), jnp.int32),
            pltpu.VMEM((2, L, DW), jnp.int32),
            pltpu.VMEM((BPW, 2 * D), jnp.float32),
            pltpu.SemaphoreType.DMA,
            pltpu.SemaphoreType.DMA,
        ],
        compiler_params=pltpu.CompilerParams(use_tc_tiling_on_sc=False,
                                             needs_layout_passes=False),
    )
    def k(x_hbm, emb_hbm, out_hbm, idx_v, rows_v, out_v, sem0, sem1):
        wid = lax.axis_index("s") * NC + lax.axis_index("c")
        base = wid * BPW
        pltpu.sync_copy(x_hbm.at[pl.ds(base, BPW)], idx_v)
        sems = (sem0, sem1)

        def start_gather(row, buf):
            sem = sems[buf]
            pltpu.make_async_copy(
                emb_hbm.at[idx_v.at[row, pl.ds(0, 128)]],
                rows_v.at[buf, pl.ds(0, 128)], sem).start()
            pltpu.make_async_copy(
                emb_hbm.at[idx_v.at[row, pl.ds(128, L - 128)]],
                rows_v.at[buf, pl.ds(128, L - 128)], sem).start()

        def wait_gather(buf):
            sem = sems[buf]
            pltpu.make_async_copy(
                emb_hbm.at[pl.ds(0, 128)], rows_v.at[buf, pl.ds(0, 128)],
                sem).wait()
            pltpu.make_async_copy(
                emb_hbm.at[pl.ds(0, L - 128)],
                rows_v.at[buf, pl.ds(128, L - 128)], sem).wait()

        def reduce_row(row, buf):
            neg = jnp.full((2 * LANES,), -3.38e38, dtype=jnp.bfloat16)
            zero = jnp.zeros((LANES,), dtype=jnp.float32)
            init = (zero, zero, zero, zero, neg, neg)

            def red_body(i, acc):
                s0, s1, s2, s3, ma, mb = acc
                w0 = rows_v[buf, i, pl.ds(0, LANES)]
                w1 = rows_v[buf, i, pl.ds(LANES, LANES)]
                bf0 = plsc.bitcast(w0, jnp.bfloat16)
                bf1 = plsc.bitcast(w1, jnp.bfloat16)
                v0, v1 = plsc.unpack(bf0,
                                     format=plsc.PackFormat.INTERLEAVED,
                                     preferred_element_type=jnp.float32)
                v2, v3 = plsc.unpack(bf1,
                                     format=plsc.PackFormat.INTERLEAVED,
                                     preferred_element_type=jnp.float32)
                return (s0 + v0, s1 + v1, s2 + v2, s3 + v3,
                        jnp.maximum(ma, bf0), jnp.maximum(mb, bf1))

            s0, s1, s2, s3, ma, mb = lax.fori_loop(
                0, L, red_body, init, unroll=4)
            m0, m1 = plsc.unpack(ma, format=plsc.PackFormat.INTERLEAVED,
                                 preferred_element_type=jnp.float32)
            m2, m3 = plsc.unpack(mb, format=plsc.PackFormat.INTERLEAVED,
                                 preferred_element_type=jnp.float32)
            inv = jnp.float32(1.0 / L)
            out_v[row, pl.ds(0, LANES)] = s0 * inv
            out_v[row, pl.ds(LANES, LANES)] = s1 * inv
            out_v[row, pl.ds(2 * LANES, LANES)] = s2 * inv
            out_v[row, pl.ds(3 * LANES, LANES)] = s3 * inv
            out_v[row, pl.ds(D, LANES)] = m0
            out_v[row, pl.ds(D + LANES, LANES)] = m1
            out_v[row, pl.ds(D + 2 * LANES, LANES)] = m2
            out_v[row, pl.ds(D + 3 * LANES, LANES)] = m3

        start_gather(0, 0)
        start_gather(1, 1)

        def pair_body(i, carry):
            row = 2 * i
            for buf in (0, 1):
                wait_gather(buf)
                reduce_row(row + buf, buf)

                @pl.when(row + buf + 2 < BPW)
                def _():
                    start_gather(row + buf + 2, buf)
            return carry

        lax.fori_loop(0, BPW // 2, pair_body, 0)
        pltpu.sync_copy(out_v, out_hbm.at[pl.ds(base, BPW)])

    return k(x, emb)


def _mlp_tc_body(p_ref, w1_ref, b1_ref, w2_ref, b2_ref, w3_ref, b3_ref,
                 o_ref):
    p = p_ref[...]
    h = lax.dot_general(p, w1_ref[...], (((1,), (1,)), ((), ())),
                        preferred_element_type=jnp.float32)
    h = jnp.maximum(h + b1_ref[...], 0.0)
    h2 = lax.dot_general(h, w2_ref[...], (((1,), (1,)), ((), ())),
                         preferred_element_type=jnp.float32)
    h2 = jnp.maximum(h2 + b2_ref[...], 0.0)
    o = lax.dot_general(h2, w3_ref[...], (((1,), (1,)), ((), ())),
                        preferred_element_type=jnp.float32)
    o_ref[...] = jax.nn.sigmoid(o + b3_ref[...])


def _mlp_tc(pooled, W1, b1, W2, b2, W3, b3):
    W3p = jnp.zeros((128, 64), jnp.float32).at[0].set(W3[0])
    b3p = jnp.zeros((1, 128), jnp.float32).at[0, 0].set(b3[0])
    out = pl.pallas_call(
        _mlp_tc_body,
        out_shape=jax.ShapeDtypeStruct((B, 128), jnp.float32),
    )(pooled, W1, b1.reshape(1, 128), W2, b2.reshape(1, 64), W3p, b3p)
    return out[:, :1]


import numpy as _np

_PERM64 = _np.concatenate([_np.arange(0, 32, 2), _np.arange(1, 32, 2),
                           _np.arange(32, 64, 2), _np.arange(33, 64, 2)])
_PERM128 = _np.concatenate([_PERM64, _PERM64 + 64])


def kernel(x, emb, W1, b1, W2, b2, W3, b3):
    tail_lin = emb[NBLK * VB:, :].reshape(64 * D)
    emb_lin = _format_sc(emb.T, tail_lin).reshape(1000000, DW)
    pooled = _pool_sc(x.astype(jnp.int32), emb_lin)
    return _mlp_tc(pooled, W1[:, _PERM128], b1, W2, b2, W3, b3)


# speedup vs baseline: 4.1747x; 1.0817x over previous
import functools

import jax
import jax.numpy as jnp
from jax import lax
from jax.experimental import pallas as pl
from jax.experimental.pallas import tpu as pltpu
from jax.experimental.pallas import tpu_sc as plsc

B = 4096
L = 200
D = 64
NC = 2
NS = 16
NW = NC * NS
BPW = B // NW
LANES = 16
NCH = D // LANES


VB = 128
NBLK = VOCAB_MAIN = 999936 // VB
BLK_PER_W = 245


DW = D // 2


def _format_sc(embT, tail_lin):
    V = 1000000
    mesh = plsc.VectorSubcoreMesh(core_axis_name="c", subcore_axis_name="s")

    @functools.partial(
        pl.kernel,
        mesh=mesh,
        out_type=jax.ShapeDtypeStruct((V * DW,), jnp.int32),
        scratch_types=[
            pltpu.VMEM((D, VB), jnp.float32),
            pltpu.VMEM((D, VB), jnp.float32),
            pltpu.VMEM((DW * VB,), jnp.int32),
            pltpu.VMEM((DW * VB,), jnp.int32),
            pltpu.VMEM((64 * D,), jnp.float32),
            pltpu.SemaphoreType.DMA,
            pltpu.SemaphoreType.DMA,
            pltpu.SemaphoreType.DMA,
            pltpu.SemaphoreType.DMA,
        ],
        compiler_params=pltpu.CompilerParams(use_tc_tiling_on_sc=True,
                                             needs_layout_passes=False),
    )
    def k(embT_hbm, tail_hbm, out_hbm, in_v0, in_v1, out_v0, out_v1,
          tail_v, si0, si1, so0, so1):
        wid = lax.axis_index("s") * NC + lax.axis_index("c")
        base_blk = wid * BLK_PER_W
        ins = (in_v0, in_v1)
        outs = (out_v0, out_v1)
        sin = (si0, si1)
        sout = (so0, so1)
        lane = jnp.arange(16, dtype=jnp.int32)
        perms = [(lane + k) & 15 for k in range(16)]
        perms2 = [p * 2 for p in perms]
        outbase = [p + lane * DW for p in perms]

        def start_in(t, buf):
            blk = base_blk + t

            @pl.when((blk < NBLK) & (t < BLK_PER_W))
            def _():
                pltpu.make_async_copy(
                    embT_hbm.at[:, pl.ds(blk * VB, VB)], ins[buf],
                    sin[buf]).start()

        def wait_in(buf):
            pltpu.make_async_copy(
                embT_hbm.at[:, pl.ds(0, VB)], ins[buf], sin[buf]).wait()

        def transpose_blk(buf):
            def vg_body(vg, carry):
                ln, pr2, ob = carry[0], carry[1:17], carry[17:]
                cols = ln + vg * 16
                for jg in range(2):
                    rows = [pr2[kk] + (32 * jg) for kk in range(16)]
                    va = [plsc.load_gather(ins[buf], [rows[kk], cols])
                          for kk in range(16)]
                    vb = [plsc.load_gather(ins[buf], [rows[kk] + 1, cols])
                          for kk in range(16)]
                    for kk in range(16):
                        w = plsc.bitcast(
                            plsc.pack(va[kk], vb[kk],
                                      format=plsc.PackFormat.INTERLEAVED),
                            jnp.int32)
                        plsc.store_scatter(
                            outs[buf],
                            [ob[kk] + (vg * 16 * DW + 16 * jg)], w)
                return carry
            lax.fori_loop(0, VB // 16, vg_body, (lane, *perms2, *outbase))

        def start_out(t, buf):
            blk = base_blk + t
            pltpu.make_async_copy(
                outs[buf], out_hbm.at[pl.ds(blk * VB * DW, VB * DW)],
                sout[buf]).start()

        def wait_out(buf):
            pltpu.make_async_copy(
                outs[buf], out_hbm.at[pl.ds(0, VB * DW)], sout[buf]).wait()

        start_in(0, 0)
        start_in(1, 1)

        def pair_body(i, carry):
            t = 2 * i
            for buf in (0, 1):
                tt = t + buf

                @pl.when((base_blk + tt < NBLK) & (tt < BLK_PER_W))
                def _():
                    @pl.when(tt >= 2)
                    def _():
                        wait_out(buf)
                    wait_in(buf)
                    transpose_blk(buf)
                    start_out(tt, buf)
                    start_in(tt + 2, buf)
            return carry

        lax.fori_loop(0, (BLK_PER_W + 1) // 2, pair_body, 0)

        @pl.when(base_blk < NBLK)
        def _():
            wait_out(0)

        @pl.when(base_blk + 1 < NBLK)
        def _():
            wait_out(1)

        @pl.when(wid == 0)
        def _():
            pltpu.sync_copy(tail_hbm, tail_v)

            def tvg_body(vg, carry):
                for jg in range(2):
                    for kk in range(16):
                        idx_a = lane * D + (vg * 16 * D + 32 * jg
                                            ) + perms2[kk]
                        va = plsc.load_gather(tail_v, [idx_a])
                        vb = plsc.load_gather(tail_v, [idx_a + 1])
                        w = plsc.bitcast(
                            plsc.pack(va, vb,
                                      format=plsc.PackFormat.INTERLEAVED),
                            jnp.int32)
                        oidx = lane * DW + perms[kk] + (vg * 16 * DW
                                                        + 16 * jg)
                        plsc.store_scatter(out_v0, [oidx], w)
                return carry
            lax.fori_loop(0, 4, tvg_body, 0)
            pltpu.sync_copy(out_v0.at[pl.ds(0, 64 * DW)],
                            out_hbm.at[pl.ds(NBLK * VB * DW, 64 * DW)])

    return k(embT, tail_lin)


def _pool_sc(x, emb):
    mesh = plsc.VectorSubcoreMesh(core_axis_name="c", subcore_axis_name="s")

    @functools.partial(
        pl.kernel,
        mesh=mesh,
        out_type=jax.ShapeDtypeStruct((B, 2 * D), jnp.float32),
        scratch_types=[
            pltpu.VMEM((BPW, L), jnp.int32),
            pltpu.VMEM((3, L, DW), jnp.int32),
            pltpu.VMEM((BPW, 2 * D), jnp.float32),
            pltpu.SemaphoreType.DMA,
            pltpu.SemaphoreType.DMA,
            pltpu.SemaphoreType.DMA,
        ],
        compiler_params=pltpu.CompilerParams(use_tc_tiling_on_sc=False,
                                             needs_layout_passes=False),
    )
    def k(x_hbm, emb_hbm, out_hbm, idx_v, rows_v, out_v, sem0, sem1, sem2):
        wid = lax.axis_index("s") * NC + lax.axis_index("c")
        base = wid * BPW
        pltpu.sync_copy(x_hbm.at[pl.ds(base, BPW)], idx_v)
        sems = (sem0, sem1, sem2)

        def start_gather(row, buf):
            sem = sems[buf]
            pltpu.make_async_copy(
                emb_hbm.at[idx_v.at[row, pl.ds(0, 128)]],
                rows_v.at[buf, pl.ds(0, 128)], sem).start()
            pltpu.make_async_copy(
                emb_hbm.at[idx_v.at[row, pl.ds(128, L - 128)]],
                rows_v.at[buf, pl.ds(128, L - 128)], sem).start()

        def wait_gather(buf):
            sem = sems[buf]
            pltpu.make_async_copy(
                emb_hbm.at[pl.ds(0, 128)], rows_v.at[buf, pl.ds(0, 128)],
                sem).wait()
            pltpu.make_async_copy(
                emb_hbm.at[pl.ds(0, L - 128)],
                rows_v.at[buf, pl.ds(128, L - 128)], sem).wait()

        def reduce_row(row, buf):
            neg = jnp.full((2 * LANES,), -3.38e38, dtype=jnp.bfloat16)
            zero = jnp.zeros((LANES,), dtype=jnp.float32)
            init = (zero, zero, zero, zero, neg, neg)

            def red_body(i, acc):
                s0, s1, s2, s3, ma, mb = acc
                w0 = rows_v[buf, i, pl.ds(0, LANES)]
                w1 = rows_v[buf, i, pl.ds(LANES, LANES)]
                bf0 = plsc.bitcast(w0, jnp.bfloat16)
                bf1 = plsc.bitcast(w1, jnp.bfloat16)
                v0, v1 = plsc.unpack(bf0,
                                     format=plsc.PackFormat.INTERLEAVED,
                                     preferred_element_type=jnp.float32)
                v2, v3 = plsc.unpack(bf1,
                                     format=plsc.PackFormat.INTERLEAVED,
                                     preferred_element_type=jnp.float32)
                return (s0 + v0, s1 + v1, s2 + v2, s3 + v3,
                        jnp.maximum(ma, bf0), jnp.maximum(mb, bf1))

            s0, s1, s2, s3, ma, mb = lax.fori_loop(
                0, L, red_body, init, unroll=4)
            m0, m1 = plsc.unpack(ma, format=plsc.PackFormat.INTERLEAVED,
                                 preferred_element_type=jnp.float32)
            m2, m3 = plsc.unpack(mb, format=plsc.PackFormat.INTERLEAVED,
                                 preferred_element_type=jnp.float32)
            inv = jnp.float32(1.0 / L)
            out_v[row, pl.ds(0, LANES)] = s0 * inv
            out_v[row, pl.ds(LANES, LANES)] = s1 * inv
            out_v[row, pl.ds(2 * LANES, LANES)] = s2 * inv
            out_v[row, pl.ds(3 * LANES, LANES)] = s3 * inv
            out_v[row, pl.ds(D, LANES)] = m0
            out_v[row, pl.ds(D + LANES, LANES)] = m1
            out_v[row, pl.ds(D + 2 * LANES, LANES)] = m2
            out_v[row, pl.ds(D + 3 * LANES, LANES)] = m3

        start_gather(0, 0)
        start_gather(1, 1)
        start_gather(2, 2)

        def tri_body(i, carry):
            row = 3 * i
            for buf in (0, 1, 2):
                r = row + buf

                @pl.when(r < BPW)
                def _():
                    wait_gather(buf)
                    reduce_row(r, buf)

                    @pl.when(r + 3 < BPW)
                    def _():
                        start_gather(r + 3, buf)
            return carry

        lax.fori_loop(0, (BPW + 2) // 3, tri_body, 0)
        pltpu.sync_copy(out_v, out_hbm.at[pl.ds(base, BPW)])

    return k(x, emb)


def _mlp_tc_body(p_ref, w1_ref, b1_ref, w2_ref, b2_ref, w3_ref, b3_ref,
                 o_ref):
    p = p_ref[...]
    h = lax.dot_general(p, w1_ref[...], (((1,), (1,)), ((), ())),
                        preferred_element_type=jnp.float32)
    h = jnp.maximum(h + b1_ref[...], 0.0)
    h2 = lax.dot_general(h, w2_ref[...], (((1,), (1,)), ((), ())),
                         preferred_element_type=jnp.float32)
    h2 = jnp.maximum(h2 + b2_ref[...], 0.0)
    o = lax.dot_general(h2, w3_ref[...], (((1,), (1,)), ((), ())),
                        preferred_element_type=jnp.float32)
    o_ref[...] = jax.nn.sigmoid(o + b3_ref[...])


def _mlp_tc(pooled, W1, b1, W2, b2, W3, b3):
    W3p = jnp.zeros((128, 64), jnp.float32).at[0].set(W3[0])
    b3p = jnp.zeros((1, 128), jnp.float32).at[0, 0].set(b3[0])
    out = pl.pallas_call(
        _mlp_tc_body,
        out_shape=jax.ShapeDtypeStruct((B, 128), jnp.float32),
    )(pooled, W1, b1.reshape(1, 128), W2, b2.reshape(1, 64), W3p, b3p)
    return out[:, :1]


import numpy as _np

_PERM64 = _np.concatenate([_np.arange(0, 32, 2), _np.arange(1, 32, 2),
                           _np.arange(32, 64, 2), _np.arange(33, 64, 2)])
_PERM128 = _np.concatenate([_PERM64, _PERM64 + 64])


def kernel(x, emb, W1, b1, W2, b2, W3, b3):
    tail_lin = emb[NBLK * VB:, :].reshape(64 * D)
    emb_lin = _format_sc(emb.T, tail_lin).reshape(1000000, DW)
    pooled = _pool_sc(x.astype(jnp.int32), emb_lin)
    return _mlp_tc(pooled, W1[:, _PERM128], b1, W2, b2, W3, b3)
